# baseline (speedup 1.0000x reference)
import functools

import jax
import jax.numpy as jnp
from jax import lax
from jax.experimental import pallas as pl
from jax.experimental.pallas import tpu as pltpu
from jax.experimental.pallas import tpu_sc as plsc

B = 16384
L = 50
EMB = 32
BP = 129

NC = 2
NS = 16
NW = NC * NS

TBG = 4
C = TBG * 128
N_UNITS = L * (B // 128) // TBG
U_PER_W = N_UNITS // NW
I_PER_W = U_PER_W * C

_mesh = plsc.VectorSubcoreMesh(core_axis_name="c", subcore_axis_name="s")


@functools.partial(
    pl.kernel,
    mesh=_mesh,
    compiler_params=pltpu.CompilerParams(
        use_tc_tiling_on_sc=False, needs_layout_passes=False
    ),
    out_type=jax.ShapeDtypeStruct((L, EMB // 8, B // 128, 8, 128), jnp.float32),
    scratch_types=[
        pltpu.VMEM((I_PER_W,), jnp.int32),
        pltpu.VMEM((C, EMB), jnp.float32),
        pltpu.VMEM((C, EMB), jnp.float32),
        pltpu.VMEM((EMB // 8, TBG, 8, BP), jnp.float32),
        pltpu.VMEM((EMB // 8, TBG, 8, BP), jnp.float32),
        pltpu.SemaphoreType.DMA,
        pltpu.SemaphoreType.DMA,
        pltpu.SemaphoreType.DMA,
        pltpu.SemaphoreType.DMA,
    ],
)
def _embed_relu(idx_hbm, table_hbm, out_hbm,
                idx_all, rows0, rows1, tile0, tile1,
                gsem0, gsem1, wsem0, wsem1):
    wid = lax.axis_index("s") * NC + lax.axis_index("c")
    lane = lax.iota(jnp.int32, 16)
    te_lo = jax.lax.shift_right_logical(lane, 3)
    te_hi = te_lo + 2
    ee_v = lane & 7
    rows_b = (rows0, rows1)
    tile_b = (tile0, tile1)
    gsem_b = (gsem0, gsem1)
    wsem_b = (wsem0, wsem1)

    pltpu.sync_copy(idx_hbm.at[pl.ds(wid * I_PER_W, I_PER_W)], idx_all)

    def start_gather(u, b):
        pltpu.async_copy(
            table_hbm.at[idx_all.at[pl.ds(u * C, C)]], rows_b[b], gsem_b[b]
        )

    def tile_src(b):
        return tile_b[b].at[:, :, :, pl.ds(0, 128)]

    start_gather(0, 0)

    def unit_body(i, carry):
        for b in range(2):
            u = i * 2 + b
            nb = 1 - b

            @pl.when(u + 1 < U_PER_W)
            def _():
                @pl.when(u >= 1)
                def _():
                    pltpu.make_async_copy(
                        tile_src(nb), out_hbm.at[0, :, pl.ds(0, TBG)], wsem_b[nb]
                    ).wait()

                start_gather(u + 1, nb)

            pltpu.make_async_copy(
                table_hbm.at[idx_all.at[pl.ds(0, C)]], rows_b[b], gsem_b[b]
            ).wait()

            for j in range(TBG):
                j_v = jnp.full((16,), j, jnp.int32)

                def row_body(rq, bb_v, j=j, j_v=j_v):
                    for s in range(4):
                        r = j * 128 + rq * 4 + s
                        v0 = rows_b[b][r, pl.ds(0, 16)]
                        v1 = rows_b[b][r, pl.ds(16, 16)]
                        plsc.store_scatter(
                            tile_b[b], [te_lo, j_v, ee_v, bb_v], jnp.maximum(v0, 0.0)
                        )
                        plsc.store_scatter(
                            tile_b[b], [te_hi, j_v, ee_v, bb_v], jnp.maximum(v1, 0.0)
                        )
                        bb_v = bb_v + 1
                    return bb_v

                lax.fori_loop(0, 32, row_body, jnp.zeros((16,), jnp.int32))

            unit = wid * U_PER_W + u
            l = unit // (B // 128 // TBG)
            tb0 = (unit % (B // 128 // TBG)) * TBG
            pltpu.async_copy(
                tile_src(b), out_hbm.at[l, :, pl.ds(tb0, TBG)], wsem_b[b]
            )
        return carry

    lax.fori_loop(0, U_PER_W // 2, unit_body, 0)
    pltpu.make_async_copy(tile_src(0), out_hbm.at[0, :, pl.ds(0, TBG)], wsem0).wait()
    pltpu.make_async_copy(tile_src(1), out_hbm.at[0, :, pl.ds(0, TBG)], wsem1).wait()


def kernel(inp, table):
    idx = inp.T.reshape(L * B).astype(jnp.int32)
    out5 = _embed_relu(idx, table)
    out = out5.transpose(2, 4, 0, 1, 3)
    return out.reshape(B, L, EMB)
